# Initial kernel scaffold; baseline (speedup 1.0000x reference)
import jax
import jax.numpy as jnp
from jax.experimental import pallas as pl


def kernel(user_emb, edge_index, W1, b1, W2, b2, W3, b3):
    raise NotImplementedError("write your pallas kernel here")


# speedup vs baseline: 12.0427x; 12.0427x over previous
import functools

import jax
import jax.numpy as jnp
from jax import lax
from jax.experimental import pallas as pl
from jax.experimental.pallas import tpu as pltpu
from jax.experimental.pallas import tpu_sc as plsc

NUM_CORES = 2
NUM_SUBCORES = 16
NUM_TILES = NUM_CORES * NUM_SUBCORES
CHUNK = 128


def _mesh():
    return plsc.VectorSubcoreMesh(core_axis_name="c", subcore_axis_name="s")


def _make_degree_kernel(n_pad, rows_tot):
    rows_per_tile = rows_tot // NUM_SUBCORES
    rpn = n_pad // NUM_SUBCORES

    def body(dst2d, zeros1d, deg_out, dstv, onesv, deg_sh):
        c = lax.axis_index("c")
        s = lax.axis_index("s")

        @pl.when(c == 0)
        def _():
            pltpu.sync_copy(zeros1d, deg_sh.at[pl.ds(s * rpn, rpn)])
            for j in range(8):
                onesv[pl.ds(j * 16, 16)] = jnp.ones((16,), jnp.float32)
            plsc.subcore_barrier()
            pltpu.sync_copy(dst2d.at[pl.ds(s * rows_per_tile, rows_per_tile)],
                            dstv)

            def step(r, carry):
                pltpu.sync_copy(onesv, deg_sh.at[dstv.at[r]], add=True)
                return carry

            lax.fori_loop(0, rows_per_tile, step, 0)
            plsc.subcore_barrier()
            pltpu.sync_copy(deg_sh.at[pl.ds(s * rpn, rpn)],
                            deg_out.at[pl.ds(s * rpn, rpn)])

    return pl.kernel(
        body,
        out_type=jax.ShapeDtypeStruct((n_pad,), jnp.float32),
        mesh=_mesh(),
        scratch_types=[
            pltpu.VMEM((rows_per_tile, CHUNK), jnp.int32),
            pltpu.VMEM((CHUNK,), jnp.float32),
            pltpu.VMEM_SHARED((n_pad,), jnp.float32),
        ],
    )


def _make_prop_kernel(n_pad, rows_tot):
    chunks_per_tile = rows_tot // NUM_TILES
    rpn = n_pad // NUM_SUBCORES

    def body(u_hbm, src2d, dst2d, zeros2d, parts_out, srcv, dstv, rows,
             acc_sh, gsem):
        c = lax.axis_index("c")
        s = lax.axis_index("s")
        wid = c * NUM_SUBCORES + s

        pltpu.sync_copy(zeros2d, acc_sh.at[pl.ds(s * rpn, rpn)])
        base = wid * chunks_per_tile
        pltpu.sync_copy(src2d.at[pl.ds(base, chunks_per_tile)], srcv)
        pltpu.sync_copy(dst2d.at[pl.ds(base, chunks_per_tile)], dstv)
        plsc.subcore_barrier()

        def step(ch, carry):
            pltpu.async_copy(u_hbm.at[srcv.at[ch]], rows, gsem).wait()
            pltpu.sync_copy(rows, acc_sh.at[dstv.at[ch]], add=True)
            return carry

        lax.fori_loop(0, chunks_per_tile, step, 0)
        plsc.subcore_barrier()
        pltpu.sync_copy(acc_sh.at[pl.ds(s * rpn, rpn)],
                        parts_out.at[c, pl.ds(s * rpn, rpn)])

    return pl.kernel(
        body,
        out_type=jax.ShapeDtypeStruct((NUM_CORES, n_pad, 128), jnp.float32),
        mesh=_mesh(),
        scratch_types=[
            pltpu.VMEM((chunks_per_tile, CHUNK), jnp.int32),
            pltpu.VMEM((chunks_per_tile, CHUNK), jnp.int32),
            pltpu.VMEM((CHUNK, 128), jnp.float32),
            pltpu.VMEM_SHARED((n_pad, 128), jnp.float32),
            pltpu.SemaphoreType.DMA,
        ],
    )


_BLK = 256


def _dis(degb):
    return jnp.where(degb > 0.0, lax.rsqrt(degb), 0.0)


def _scale0_body(x_ref, deg_ref, o_ref):
    o_ref[...] = _dis(deg_ref[...]) * x_ref[...]


def _mid_body(p0_ref, p1_ref, deg_ref, o_ref):
    d = _dis(deg_ref[...])
    o_ref[...] = d * d * (p0_ref[...] + p1_ref[...])


def _layer_body(p0_ref, p1_ref, deg_ref, w_ref, b_ref, o_ref, *, final):
    d = _dis(deg_ref[...])
    t = d * (p0_ref[...] + p1_ref[...])
    y = lax.dot_general(t, w_ref[...], (((1,), (1,)), ((), ())),
                        preferred_element_type=jnp.float32,
                        precision=lax.Precision.HIGHEST)
    y = y + b_ref[0:1, :]
    o_ref[...] = y if final else d * y


def _row_spec():
    return pl.BlockSpec((_BLK, 128), lambda i: (i, 0))


def _tc_call(body, n_pad, n_in):
    grid = (n_pad // _BLK,)
    return pl.pallas_call(
        body,
        grid=grid,
        in_specs=[_row_spec()] * n_in,
        out_specs=_row_spec(),
        out_shape=jax.ShapeDtypeStruct((n_pad, 128), jnp.float32),
    )


def _tc_layer_call(body, n_pad):
    grid = (n_pad // _BLK,)
    return pl.pallas_call(
        body,
        grid=grid,
        in_specs=[_row_spec(), _row_spec(), _row_spec(),
                  pl.BlockSpec((128, 128), lambda i: (0, 0)),
                  pl.BlockSpec((8, 128), lambda i: (0, 0))],
        out_specs=_row_spec(),
        out_shape=jax.ShapeDtypeStruct((n_pad, 128), jnp.float32),
    )


def kernel(user_emb, edge_index, W1, b1, W2, b2, W3, b3):
    n, d_feat = user_emb.shape
    e = edge_index.shape[1]
    e_tot = e + n

    n_pad = ((n + NUM_TILES * 8 - 1) // (NUM_TILES * 8)) * (NUM_TILES * 8)
    chunks_per_tile = (e_tot + NUM_TILES * CHUNK - 1) // (NUM_TILES * CHUNK)
    chunks_per_tile = ((chunks_per_tile + 7) // 8) * 8
    e_pad = NUM_TILES * chunks_per_tile * CHUNK
    rows_tot = e_pad // CHUNK
    n_spare = max(n_pad - n, 1)

    loop = jnp.arange(n, dtype=jnp.int32)
    pad_idx = n + jnp.arange(e_pad - e_tot, dtype=jnp.int32) % n_spare
    src = jnp.concatenate([edge_index[0].astype(jnp.int32), loop, pad_idx])
    dst = jnp.concatenate([edge_index[1].astype(jnp.int32), loop, pad_idx])
    src2d = src.reshape(rows_tot, CHUNK)
    dst2d = dst.reshape(rows_tot, CHUNK)

    x0 = jnp.zeros((n_pad, 128), jnp.float32).at[:n, :d_feat].set(user_emb)
    zeros1d = jnp.zeros((n_pad // NUM_SUBCORES,), jnp.float32)
    zeros2d = jnp.zeros((n_pad // NUM_SUBCORES, 128), jnp.float32)

    deg = _make_degree_kernel(n_pad, rows_tot)(dst2d, zeros1d)
    degb = jnp.broadcast_to(deg[:, None], (n_pad, 128))

    prop = _make_prop_kernel(n_pad, rows_tot)
    scale0 = _tc_call(_scale0_body, n_pad, 2)
    mid = _tc_call(_mid_body, n_pad, 3)
    layer = _tc_layer_call(functools.partial(_layer_body, final=False), n_pad)
    last = _tc_layer_call(functools.partial(_layer_body, final=True), n_pad)

    b1_8 = jnp.broadcast_to(b1[None, :], (8, 128))
    b2_8 = jnp.broadcast_to(b2[None, :], (8, 128))
    b3_8 = jnp.broadcast_to(b3[None, :], (8, 128))

    u = scale0(x0, degb)
    for li, (w, b8) in enumerate(((W1, b1_8), (W2, b2_8), (W3, b3_8))):
        for k in range(3):
            parts = prop(u, src2d, dst2d, zeros2d)
            if k < 2:
                u = mid(parts[0], parts[1], degb)
            elif li < 2:
                u = layer(parts[0], parts[1], degb, w, b8)
            else:
                u = last(parts[0], parts[1], degb, w, b8)
    return u[:n, :d_feat]


# speedup vs baseline: 17.4023x; 1.4450x over previous
import functools

import jax
import jax.numpy as jnp
from jax import lax
from jax.experimental import pallas as pl
from jax.experimental.pallas import tpu as pltpu
from jax.experimental.pallas import tpu_sc as plsc

NUM_CORES = 2
NUM_SUBCORES = 16
NUM_TILES = NUM_CORES * NUM_SUBCORES
CHUNK = 128


def _mesh():
    return plsc.VectorSubcoreMesh(core_axis_name="c", subcore_axis_name="s")


def _make_degree_kernel(n_pad, rows_tot):
    rows_per_tile = rows_tot // NUM_SUBCORES
    rpn = n_pad // NUM_SUBCORES

    def body(dst2d, zeros1d, deg_out, dstv, onesv, deg_sh):
        c = lax.axis_index("c")
        s = lax.axis_index("s")

        @pl.when(c == 0)
        def _():
            pltpu.sync_copy(zeros1d, deg_sh.at[pl.ds(s * rpn, rpn)])
            for j in range(8):
                onesv[pl.ds(j * 16, 16)] = jnp.ones((16,), jnp.float32)
            plsc.subcore_barrier()

            def step(r8, carry):
                pltpu.sync_copy(
                    dst2d.at[pl.ds(s * rows_per_tile + r8 * 8, 8)], dstv)
                for r in range(8):
                    pltpu.sync_copy(onesv, deg_sh.at[dstv.at[r]], add=True)
                return carry

            lax.fori_loop(0, rows_per_tile // 8, step, 0)
            plsc.subcore_barrier()
            pltpu.sync_copy(deg_sh.at[pl.ds(s * rpn, rpn)],
                            deg_out.at[pl.ds(s * rpn, rpn)])

    return pl.kernel(
        body,
        out_type=jax.ShapeDtypeStruct((n_pad,), jnp.float32),
        mesh=_mesh(),
        scratch_types=[
            pltpu.VMEM((8, CHUNK), jnp.int32),
            pltpu.VMEM((CHUNK,), jnp.float32),
            pltpu.VMEM_SHARED((n_pad,), jnp.float32),
        ],
    )


_NBUF = 2
_GRP = 8


def _make_prop_kernel(n_pad, rows_tot):
    chunks_per_tile = rows_tot // NUM_TILES
    assert chunks_per_tile % _GRP == 0
    n_grp = chunks_per_tile // _GRP
    rpn = n_pad // NUM_SUBCORES

    def body(u_hbm, src2d, dst2d, zeros2d, parts_out, *scratch):
        srcg = scratch[:2]
        dstg = scratch[2:4]
        isems = scratch[4:6]
        rows = scratch[6:6 + _NBUF]
        gsems = scratch[6 + _NBUF:6 + 2 * _NBUF]
        ssem = scratch[6 + 2 * _NBUF]
        acc_sh = scratch[6 + 2 * _NBUF + 1]
        c = lax.axis_index("c")
        s = lax.axis_index("s")
        wid = c * NUM_SUBCORES + s
        base = wid * chunks_per_tile

        def igstart(gi):
            b = gi % 2
            sl = pl.ds(base + _GRP * gi, _GRP)
            pltpu.async_copy(src2d.at[sl], srcg[b], isems[b])
            pltpu.async_copy(dst2d.at[sl], dstg[b], isems[b])

        def igwait(gi):
            b = gi % 2
            sl = pl.ds(base + _GRP * gi, _GRP)
            pltpu.make_async_copy(src2d.at[sl], srcg[b], isems[b]).wait()
            pltpu.make_async_copy(dst2d.at[sl], dstg[b], isems[b]).wait()

        def gstart(ch):
            gi, j = divmod(ch, _GRP)
            pltpu.async_copy(u_hbm.at[srcg[gi % 2].at[j]], rows[ch % _NBUF],
                             gsems[ch % _NBUF])

        def gwait(ch):
            gi, j = divmod(ch, _GRP)
            pltpu.make_async_copy(u_hbm.at[srcg[gi % 2].at[j]],
                                  rows[ch % _NBUF],
                                  gsems[ch % _NBUF]).wait()

        def scatter(ch):
            gi, j = divmod(ch, _GRP)
            pltpu.async_copy(rows[ch % _NBUF], acc_sh.at[dstg[gi % 2].at[j]],
                             ssem, add=True).wait()

        pltpu.sync_copy(zeros2d, acc_sh.at[pl.ds(s * rpn, rpn)])
        plsc.subcore_barrier()

        igstart(0)
        if n_grp > 1:
            igstart(1)
        igwait(0)
        for b in range(_NBUF):
            gstart(b)
        for ch in range(chunks_per_tile):
            gi, j = divmod(ch, _GRP)
            if j == _GRP - 2 and gi + 1 < n_grp:
                igwait(gi + 1)
            gwait(ch)
            scatter(ch)
            if j == _GRP - 1 and gi + 2 < n_grp:
                igstart(gi + 2)
            if ch + _NBUF < chunks_per_tile:
                gstart(ch + _NBUF)

        plsc.subcore_barrier()
        pltpu.sync_copy(acc_sh.at[pl.ds(s * rpn, rpn)],
                        parts_out.at[c, pl.ds(s * rpn, rpn)])

    return pl.kernel(
        body,
        out_type=jax.ShapeDtypeStruct((NUM_CORES, n_pad, 128), jnp.float32),
        mesh=_mesh(),
        scratch_types=[
            *[pltpu.VMEM((_GRP, CHUNK), jnp.int32) for _ in range(2)],
            *[pltpu.VMEM((_GRP, CHUNK), jnp.int32) for _ in range(2)],
            *[pltpu.SemaphoreType.DMA for _ in range(2)],
            *[pltpu.VMEM((CHUNK, 128), jnp.float32) for _ in range(_NBUF)],
            *[pltpu.SemaphoreType.DMA for _ in range(_NBUF)],
            pltpu.SemaphoreType.DMA,
            pltpu.VMEM_SHARED((n_pad, 128), jnp.float32),
        ],
    )


_BLK = 256


def _dis(degb):
    return jnp.where(degb > 0.0, lax.rsqrt(degb), 0.0)


def _scale0_body(x_ref, deg_ref, o_ref):
    o_ref[...] = _dis(deg_ref[...]) * x_ref[...]


def _mid_body(p0_ref, p1_ref, deg_ref, o_ref):
    d = _dis(deg_ref[...])
    o_ref[...] = d * d * (p0_ref[...] + p1_ref[...])


def _layer_body(p0_ref, p1_ref, deg_ref, w_ref, b_ref, o_ref, *, final):
    d = _dis(deg_ref[...])
    t = d * (p0_ref[...] + p1_ref[...])
    y = lax.dot_general(t, w_ref[...], (((1,), (1,)), ((), ())),
                        preferred_element_type=jnp.float32,
                        precision=lax.Precision.HIGHEST)
    y = y + b_ref[0:1, :]
    o_ref[...] = y if final else d * y


def _row_spec():
    return pl.BlockSpec((_BLK, 128), lambda i: (i, 0))


def _tc_call(body, n_pad, n_in):
    grid = (n_pad // _BLK,)
    return pl.pallas_call(
        body,
        grid=grid,
        in_specs=[_row_spec()] * n_in,
        out_specs=_row_spec(),
        out_shape=jax.ShapeDtypeStruct((n_pad, 128), jnp.float32),
    )


def _tc_layer_call(body, n_pad):
    grid = (n_pad // _BLK,)
    return pl.pallas_call(
        body,
        grid=grid,
        in_specs=[_row_spec(), _row_spec(), _row_spec(),
                  pl.BlockSpec((128, 128), lambda i: (0, 0)),
                  pl.BlockSpec((8, 128), lambda i: (0, 0))],
        out_specs=_row_spec(),
        out_shape=jax.ShapeDtypeStruct((n_pad, 128), jnp.float32),
    )


def kernel(user_emb, edge_index, W1, b1, W2, b2, W3, b3):
    n, d_feat = user_emb.shape
    e = edge_index.shape[1]
    e_tot = e + n

    n_pad = ((n + NUM_TILES * 8 - 1) // (NUM_TILES * 8)) * (NUM_TILES * 8)
    chunks_per_tile = (e_tot + NUM_TILES * CHUNK - 1) // (NUM_TILES * CHUNK)
    chunks_per_tile = ((chunks_per_tile + 7) // 8) * 8
    e_pad = NUM_TILES * chunks_per_tile * CHUNK
    rows_tot = e_pad // CHUNK
    n_spare = max(n_pad - n, 1)

    loop = jnp.arange(n, dtype=jnp.int32)
    pad_idx = n + jnp.arange(e_pad - e_tot, dtype=jnp.int32) % n_spare
    src = jnp.concatenate([edge_index[0].astype(jnp.int32), loop, pad_idx])
    dst = jnp.concatenate([edge_index[1].astype(jnp.int32), loop, pad_idx])
    src2d = src.reshape(rows_tot, CHUNK)
    dst2d = dst.reshape(rows_tot, CHUNK)

    x0 = jnp.zeros((n_pad, 128), jnp.float32).at[:n, :d_feat].set(user_emb)
    zeros1d = jnp.zeros((n_pad // NUM_SUBCORES,), jnp.float32)
    zeros2d = jnp.zeros((n_pad // NUM_SUBCORES, 128), jnp.float32)

    deg = _make_degree_kernel(n_pad, rows_tot)(dst2d, zeros1d)
    degb = jnp.broadcast_to(deg[:, None], (n_pad, 128))

    prop = _make_prop_kernel(n_pad, rows_tot)
    scale0 = _tc_call(_scale0_body, n_pad, 2)
    mid = _tc_call(_mid_body, n_pad, 3)
    layer = _tc_layer_call(functools.partial(_layer_body, final=False), n_pad)
    last = _tc_layer_call(functools.partial(_layer_body, final=True), n_pad)

    b1_8 = jnp.broadcast_to(b1[None, :], (8, 128))
    b2_8 = jnp.broadcast_to(b2[None, :], (8, 128))
    b3_8 = jnp.broadcast_to(b3[None, :], (8, 128))

    u = scale0(x0, degb)
    for li, (w, b8) in enumerate(((W1, b1_8), (W2, b2_8), (W3, b3_8))):
        for k in range(3):
            parts = prop(u, src2d, dst2d, zeros2d)
            if k < 2:
                u = mid(parts[0], parts[1], degb)
            elif li < 2:
                u = layer(parts[0], parts[1], degb, w, b8)
            else:
                u = last(parts[0], parts[1], degb, w, b8)
    return u[:n, :d_feat]


# speedup vs baseline: 18.4396x; 1.0596x over previous
import functools

import jax
import jax.numpy as jnp
from jax import lax
from jax.experimental import pallas as pl
from jax.experimental.pallas import tpu as pltpu
from jax.experimental.pallas import tpu_sc as plsc

NUM_CORES = 2
NUM_SUBCORES = 16
NUM_TILES = NUM_CORES * NUM_SUBCORES
CHUNK = 128


def _mesh():
    return plsc.VectorSubcoreMesh(core_axis_name="c", subcore_axis_name="s")


def _make_degree_kernel(n_pad, rows_tot):
    rows_per_tile = rows_tot // NUM_SUBCORES
    rpn = n_pad // NUM_SUBCORES

    def body(dst2d, zeros1d, deg_out, dstv, onesv, deg_sh):
        c = lax.axis_index("c")
        s = lax.axis_index("s")

        @pl.when(c == 0)
        def _():
            pltpu.sync_copy(zeros1d, deg_sh.at[pl.ds(s * rpn, rpn)])
            for j in range(8):
                onesv[pl.ds(j * 16, 16)] = jnp.ones((16,), jnp.float32)
            plsc.subcore_barrier()

            def step(r8, carry):
                pltpu.sync_copy(
                    dst2d.at[pl.ds(s * rows_per_tile + r8 * 8, 8)], dstv)
                for r in range(8):
                    pltpu.sync_copy(onesv, deg_sh.at[dstv.at[r]], add=True)
                return carry

            lax.fori_loop(0, rows_per_tile // 8, step, 0)
            plsc.subcore_barrier()
            pltpu.sync_copy(deg_sh.at[pl.ds(s * rpn, rpn)],
                            deg_out.at[pl.ds(s * rpn, rpn)])

    return pl.kernel(
        body,
        out_type=jax.ShapeDtypeStruct((n_pad,), jnp.float32),
        mesh=_mesh(),
        scratch_types=[
            pltpu.VMEM((8, CHUNK), jnp.int32),
            pltpu.VMEM((CHUNK,), jnp.float32),
            pltpu.VMEM_SHARED((n_pad,), jnp.float32),
        ],
    )


_NBUF = 2
_GRP = 8


def _make_prop_kernel(n_pad, rows_tot):
    chunks_per_tile = rows_tot // NUM_TILES
    assert chunks_per_tile % _GRP == 0
    n_grp = chunks_per_tile // _GRP
    rpn = n_pad // NUM_SUBCORES

    def body(u_hbm, src2d, dst2d, zeros2d, parts_out, *scratch):
        srcg = scratch[:2]
        dstg = scratch[2:4]
        isems = scratch[4:6]
        rows = scratch[6:6 + _NBUF]
        gsems = scratch[6 + _NBUF:6 + 2 * _NBUF]
        ssem = scratch[6 + 2 * _NBUF]
        acc_sh = scratch[6 + 2 * _NBUF + 1]
        c = lax.axis_index("c")
        s = lax.axis_index("s")
        wid = c * NUM_SUBCORES + s
        base = wid * chunks_per_tile

        def igstart(gi):
            b = gi % 2
            sl = pl.ds(base + _GRP * gi, _GRP)
            pltpu.async_copy(src2d.at[sl], srcg[b], isems[b])
            pltpu.async_copy(dst2d.at[sl], dstg[b], isems[b])

        def igwait(gi):
            b = gi % 2
            sl = pl.ds(base + _GRP * gi, _GRP)
            pltpu.make_async_copy(src2d.at[sl], srcg[b], isems[b]).wait()
            pltpu.make_async_copy(dst2d.at[sl], dstg[b], isems[b]).wait()

        def gstart(ch):
            gi, j = divmod(ch, _GRP)
            pltpu.async_copy(u_hbm.at[srcg[gi % 2].at[j]], rows[ch % _NBUF],
                             gsems[ch % _NBUF])

        def gwait(ch):
            gi, j = divmod(ch, _GRP)
            pltpu.make_async_copy(u_hbm.at[srcg[gi % 2].at[j]],
                                  rows[ch % _NBUF],
                                  gsems[ch % _NBUF]).wait()

        def scatter(ch):
            gi, j = divmod(ch, _GRP)
            pltpu.async_copy(rows[ch % _NBUF], acc_sh.at[dstg[gi % 2].at[j]],
                             ssem, add=True).wait()

        pltpu.sync_copy(zeros2d, acc_sh.at[pl.ds(s * rpn, rpn)])
        plsc.subcore_barrier()

        igstart(0)
        if n_grp > 1:
            igstart(1)
        igwait(0)
        for b in range(_NBUF):
            gstart(b)
        for ch in range(chunks_per_tile):
            gi, j = divmod(ch, _GRP)
            if j == _GRP - 2 and gi + 1 < n_grp:
                igwait(gi + 1)
            gwait(ch)
            scatter(ch)
            if j == _GRP - 1 and gi + 2 < n_grp:
                igstart(gi + 2)
            if ch + _NBUF < chunks_per_tile:
                gstart(ch + _NBUF)

        plsc.subcore_barrier()
        pltpu.sync_copy(acc_sh.at[pl.ds(s * rpn, rpn)],
                        parts_out.at[c, pl.ds(s * rpn, rpn)])

    return pl.kernel(
        body,
        out_type=jax.ShapeDtypeStruct((NUM_CORES, n_pad, 128), jnp.float32),
        mesh=_mesh(),
        scratch_types=[
            *[pltpu.VMEM((_GRP, CHUNK), jnp.int32) for _ in range(2)],
            *[pltpu.VMEM((_GRP, CHUNK), jnp.int32) for _ in range(2)],
            *[pltpu.SemaphoreType.DMA for _ in range(2)],
            *[pltpu.VMEM((CHUNK, 128), jnp.float32) for _ in range(_NBUF)],
            *[pltpu.SemaphoreType.DMA for _ in range(_NBUF)],
            pltpu.SemaphoreType.DMA,
            pltpu.VMEM_SHARED((n_pad, 128), jnp.float32),
        ],
    )


_BLK = 256


def _dis(degb):
    return lax.rsqrt(degb + 1.0)


def _scale0_body(x_ref, deg_ref, o_ref):
    o_ref[...] = _dis(deg_ref[...]) * x_ref[...]


def _mid_body(p0_ref, p1_ref, u_ref, deg_ref, o_ref):
    d = _dis(deg_ref[...])
    o_ref[...] = d * d * (p0_ref[...] + p1_ref[...] + u_ref[...])


def _layer_body(p0_ref, p1_ref, u_ref, deg_ref, w_ref, b_ref, o_ref, *,
                final):
    d = _dis(deg_ref[...])
    t = d * (p0_ref[...] + p1_ref[...] + u_ref[...])
    y = lax.dot_general(t, w_ref[...], (((1,), (1,)), ((), ())),
                        preferred_element_type=jnp.float32,
                        precision=lax.Precision.HIGHEST)
    y = y + b_ref[0:1, :]
    o_ref[...] = y if final else d * y


def _row_spec():
    return pl.BlockSpec((_BLK, 128), lambda i: (i, 0))


def _tc_call(body, n_pad, n_in):
    grid = (n_pad // _BLK,)
    return pl.pallas_call(
        body,
        grid=grid,
        in_specs=[_row_spec()] * n_in,
        out_specs=_row_spec(),
        out_shape=jax.ShapeDtypeStruct((n_pad, 128), jnp.float32),
    )


def _tc_layer_call(body, n_pad):
    grid = (n_pad // _BLK,)
    return pl.pallas_call(
        body,
        grid=grid,
        in_specs=[_row_spec(), _row_spec(), _row_spec(), _row_spec(),
                  pl.BlockSpec((128, 128), lambda i: (0, 0)),
                  pl.BlockSpec((8, 128), lambda i: (0, 0))],
        out_specs=_row_spec(),
        out_shape=jax.ShapeDtypeStruct((n_pad, 128), jnp.float32),
    )


def kernel(user_emb, edge_index, W1, b1, W2, b2, W3, b3):
    n, d_feat = user_emb.shape
    e = edge_index.shape[1]
    e_tot = e

    n_pad = ((n + NUM_TILES * 8 - 1) // (NUM_TILES * 8)) * (NUM_TILES * 8)
    chunks_per_tile = (e_tot + NUM_TILES * CHUNK - 1) // (NUM_TILES * CHUNK)
    chunks_per_tile = ((chunks_per_tile + 7) // 8) * 8
    e_pad = NUM_TILES * chunks_per_tile * CHUNK
    rows_tot = e_pad // CHUNK
    n_spare = max(n_pad - n, 1)

    pad_idx = n + jnp.arange(e_pad - e_tot, dtype=jnp.int32) % n_spare
    src = jnp.concatenate([edge_index[0].astype(jnp.int32), pad_idx])
    dst = jnp.concatenate([edge_index[1].astype(jnp.int32), pad_idx])
    src2d = src.reshape(rows_tot, CHUNK)
    dst2d = dst.reshape(rows_tot, CHUNK)

    x0 = jnp.zeros((n_pad, 128), jnp.float32).at[:n, :d_feat].set(user_emb)
    zeros1d = jnp.zeros((n_pad // NUM_SUBCORES,), jnp.float32)
    zeros2d = jnp.zeros((n_pad // NUM_SUBCORES, 128), jnp.float32)

    deg = _make_degree_kernel(n_pad, rows_tot)(dst2d, zeros1d)
    degb = jnp.broadcast_to(deg[:, None], (n_pad, 128))

    prop = _make_prop_kernel(n_pad, rows_tot)
    scale0 = _tc_call(_scale0_body, n_pad, 2)
    mid = _tc_call(_mid_body, n_pad, 4)
    layer = _tc_layer_call(functools.partial(_layer_body, final=False), n_pad)
    last = _tc_layer_call(functools.partial(_layer_body, final=True), n_pad)

    b1_8 = jnp.broadcast_to(b1[None, :], (8, 128))
    b2_8 = jnp.broadcast_to(b2[None, :], (8, 128))
    b3_8 = jnp.broadcast_to(b3[None, :], (8, 128))

    u = scale0(x0, degb)
    for li, (w, b8) in enumerate(((W1, b1_8), (W2, b2_8), (W3, b3_8))):
        for k in range(3):
            parts = prop(u, src2d, dst2d, zeros2d)
            if k < 2:
                u = mid(parts[0], parts[1], u, degb)
            elif li < 2:
                u = layer(parts[0], parts[1], u, degb, w, b8)
            else:
                u = last(parts[0], parts[1], u, degb, w, b8)
    return u[:n, :d_feat]
